# baseline (speedup 1.0000x reference)
import functools

import jax
import jax.numpy as jnp
from jax.experimental import pallas as pl

_DIM = 96
_NITEM = 1024
_K = 10
_CBLK = 3200
_NEG = -1e30


def _mem_kernel(x_ref, mem_ref, mt_ref, out_ref, loss_ref, *, nb):
    b = pl.program_id(0)
    j = pl.program_id(1)
    mem = mem_ref[...]
    mt = mt_ref[...]
    x = x_ref[0]

    logits = jnp.dot(mem, x, preferred_element_type=jnp.float32)

    lo = jnp.minimum(logits[: _NITEM // 2], logits[_NITEM // 2 :])
    work = jnp.maximum(logits[: _NITEM // 2], logits[_NITEM // 2 :])
    m0 = jnp.max(work, axis=0, keepdims=True)
    cur = m0
    for _ in range(_K - 1):
        work = jnp.where(work == cur,
                         jnp.where(work == lo, _NEG, lo), work)
        cur = jnp.max(work, axis=0, keepdims=True)

    inv_z = 1.0 / jnp.sum(jnp.exp(logits - m0), axis=0,
                          keepdims=True)
    att0 = jnp.where(logits >= cur,
                     jnp.exp(jnp.exp(logits - m0) * inv_z), 0.0)
    inv_d = 1.0 / jnp.sum(att0, axis=0, keepdims=True)

    out_ref[0] = jnp.dot(mt, att0, preferred_element_type=jnp.float32) * inv_d

    rows = _NITEM // nb
    @pl.when(j == 0)
    def _loss():
        chunk = mem_ref[pl.ds(b * rows, rows), :]
        cos = jnp.dot(chunk, mt, preferred_element_type=jnp.float32) * 0.5
        ii = jax.lax.broadcasted_iota(jnp.int32, (rows, _NITEM), 0) + b * rows
        jj = jax.lax.broadcasted_iota(jnp.int32, (rows, _NITEM), 1)
        part = jnp.sum(jnp.where(ii == jj, 0.0, jnp.abs(cos)),
                       axis=(0, 1), keepdims=True)

        @pl.when(b == 0)
        def _init():
            loss_ref[...] = part

        @pl.when(b > 0)
        def _acc():
            loss_ref[...] += part


def kernel(input, mempool):
    B, CH, H, W = input.shape
    hw = H * W
    hwp = ((hw + _CBLK - 1) // _CBLK) * _CBLK
    x = input.reshape(B, CH, hw)
    if hwp != hw:
        x = jnp.pad(x, ((0, 0), (0, 0), (0, hwp - hw)))
    mt = mempool.T
    out, loss = pl.pallas_call(
        functools.partial(_mem_kernel, nb=B),
        grid=(B, hwp // _CBLK),
        in_specs=[
            pl.BlockSpec((1, CH, _CBLK), lambda b, j: (b, 0, j)),
            pl.BlockSpec((_NITEM, _DIM), lambda b, j: (0, 0)),
            pl.BlockSpec((_DIM, _NITEM), lambda b, j: (0, 0)),
        ],
        out_specs=[
            pl.BlockSpec((1, CH, _CBLK), lambda b, j: (b, 0, j)),
            pl.BlockSpec((1, 1), lambda b, j: (0, 0)),
        ],
        out_shape=[
            jax.ShapeDtypeStruct((B, CH, hwp), jnp.float32),
            jax.ShapeDtypeStruct((1, 1), jnp.float32),
        ],
    )(x, mempool, mt)
    out = out[:, :, :hw] if hwp != hw else out
    return out.reshape(B, CH, H, W), loss[0, 0] / (_NITEM * _NITEM)


# speedup vs baseline: 1.3460x; 1.3460x over previous
import jax
import jax.numpy as jnp
from jax.experimental import pallas as pl

_DIM = 96
_NITEM = 1024
_K = 10
_CBLK = 3200
_NEG = -1e30


def _mem_kernel(x_ref, mem_ref, mt_ref, out_ref, loss_ref):
    b = pl.program_id(0)
    j = pl.program_id(1)
    mem = mem_ref[...]
    mt = mt_ref[...]
    x = x_ref[0]

    logits = jnp.dot(mem, x, preferred_element_type=jnp.float32)

    lo = jnp.minimum(logits[: _NITEM // 2], logits[_NITEM // 2 :])
    work = jnp.maximum(logits[: _NITEM // 2], logits[_NITEM // 2 :])
    m0 = jnp.max(work, axis=0, keepdims=True)
    cur = m0
    for _ in range(_K - 1):
        work = jnp.where(work == cur,
                         jnp.where(work == lo, _NEG, lo), work)
        cur = jnp.max(work, axis=0, keepdims=True)

    inv_z = 1.0 / jnp.sum(jnp.exp(logits - m0), axis=0,
                          keepdims=True)
    att0 = jnp.where(logits >= cur,
                     jnp.exp(jnp.exp(logits - m0) * inv_z), 0.0)
    inv_d = 1.0 / jnp.sum(att0, axis=0, keepdims=True)

    out_ref[0] = jnp.dot(mt, att0, preferred_element_type=jnp.float32) * inv_d

    @pl.when(jnp.logical_and(b == 0, j == 0))
    def _loss():
        cos = jnp.dot(mem, mt, preferred_element_type=jnp.float32) * 0.5
        ii = jax.lax.broadcasted_iota(jnp.int32, (_NITEM, _NITEM), 0)
        jj = jax.lax.broadcasted_iota(jnp.int32, (_NITEM, _NITEM), 1)
        loss_ref[...] = jnp.sum(jnp.where(ii == jj, 0.0, jnp.abs(cos)),
                                axis=(0, 1), keepdims=True)


def kernel(input, mempool):
    B, CH, H, W = input.shape
    hw = H * W
    x = input.reshape(B, CH, hw)
    mt = mempool.T
    out, loss = pl.pallas_call(
        _mem_kernel,
        grid=(B, (hw + _CBLK - 1) // _CBLK),
        in_specs=[
            pl.BlockSpec((1, CH, _CBLK), lambda b, j: (b, 0, j)),
            pl.BlockSpec((_NITEM, _DIM), lambda b, j: (0, 0)),
            pl.BlockSpec((_DIM, _NITEM), lambda b, j: (0, 0)),
        ],
        out_specs=[
            pl.BlockSpec((1, CH, _CBLK), lambda b, j: (b, 0, j)),
            pl.BlockSpec((1, 1), lambda b, j: (0, 0)),
        ],
        out_shape=[
            jax.ShapeDtypeStruct((B, CH, hw), jnp.float32),
            jax.ShapeDtypeStruct((1, 1), jnp.float32),
        ],
    )(x, mempool, mt)
    return out.reshape(B, CH, H, W), loss[0, 0] / (_NITEM * _NITEM)
